# baseline (speedup 1.0000x reference)
import jax
import jax.numpy as jnp
from jax import lax
from jax.experimental import pallas as pl
from jax.experimental.pallas import tpu as pltpu
from jax.experimental.pallas import tpu_sc as plsc

_FI = (2, 3, 0, 1, 4, 6, 5, 7, 9, 8, 11, 10, 12, 14, 13, 16, 15, 18, 17)
_REV = (4, 7, 12)

_B = 64
_J = 19
_H = 64
_W = 64
_NW = 32
_BPW = _B // _NW
_NIB = 4
_NOB = 3


def _sc_body(f0, f1, f2, o0, o1, o2, ibuf, obuf,
             isem0, isem1, isem2, isem3, osem0, osem1, osem2):
  isems = (isem0, isem1, isem2, isem3)
  osems = (osem0, osem1, osem2)
  wid = lax.axis_index("s") * 2 + lax.axis_index("c")
  bb = wid * _BPW

  items = []
  for j in range(_J):
    fij = _FI[j]
    in_rev = j in _REV
    src_a = f2 if in_rev else f1
    src_b = f1 if in_rev else f2
    items.append((f0.at[pl.ds(bb, _BPW), fij],
                  o0.at[pl.ds(bb, _BPW), j], 1))
    for c in range(2):
      sign = -1 if c == 0 else 1
      items.append((src_a.at[pl.ds(bb, _BPW), fij, c],
                    o1.at[pl.ds(bb, _BPW), j, c], sign))
      items.append((src_b.at[pl.ds(bb, _BPW), fij, c],
                    o2.at[pl.ds(bb, _BPW), j, c], sign))
  num_items = len(items)

  def rev_block(islot, oslot, sign):
    @plsc.parallel_loop(0, _H, unroll=2)
    def row(r):
      for img in range(_BPW):
        c0 = ibuf[islot, img, r, pl.ds(0, 16)]
        c1 = ibuf[islot, img, r, pl.ds(16, 16)]
        c2 = ibuf[islot, img, r, pl.ds(32, 16)]
        c3 = ibuf[islot, img, r, pl.ds(48, 16)]
        w0, w1, w2, w3 = jnp.flip(c3), jnp.flip(c2), jnp.flip(c1), jnp.flip(c0)
        if sign < 0:
          w0, w1, w2, w3 = -w0, -w1, -w2, -w3
        obuf[oslot, img, r, pl.ds(0, 16)] = w0
        obuf[oslot, img, r, pl.ds(16, 16)] = w1
        obuf[oslot, img, r, pl.ds(32, 16)] = w2
        obuf[oslot, img, r, pl.ds(48, 16)] = w3

  handles_in = {}
  handles_out = {}

  def start_gather(i):
    slot = i % _NIB
    src, _, _ = items[i]
    handles_in[i] = pltpu.async_copy(src, ibuf.at[slot], isems[slot])

  for i in range(_NIB):
    start_gather(i)
  for i in range(num_items):
    islot = i % _NIB
    oslot = i % _NOB
    _, dst, sign = items[i]
    handles_in[i].wait()
    if i >= _NOB:
      handles_out[i - _NOB].wait()
    rev_block(islot, oslot, sign)
    handles_out[i] = pltpu.async_copy(obuf.at[oslot], dst, osems[oslot])
    if i + _NIB < num_items:
      start_gather(i + _NIB)
  for i in range(num_items - _NOB, num_items):
    handles_out[i].wait()


@jax.jit
def kernel(field0, field1, field2):
  mesh = plsc.VectorSubcoreMesh(
      core_axis_name="c", subcore_axis_name="s", num_cores=2, num_subcores=16)
  run = pl.kernel(
      _sc_body,
      out_type=(
          jax.ShapeDtypeStruct(field0.shape, jnp.float32),
          jax.ShapeDtypeStruct(field1.shape, jnp.float32),
          jax.ShapeDtypeStruct(field2.shape, jnp.float32),
      ),
      mesh=mesh,
      scratch_types=[
          pltpu.VMEM((_NIB, _BPW, _H, _W), jnp.float32),
          pltpu.VMEM((_NOB, _BPW, _H, _W), jnp.float32),
          pltpu.SemaphoreType.DMA,
          pltpu.SemaphoreType.DMA,
          pltpu.SemaphoreType.DMA,
          pltpu.SemaphoreType.DMA,
          pltpu.SemaphoreType.DMA,
          pltpu.SemaphoreType.DMA,
          pltpu.SemaphoreType.DMA,
      ],
      compiler_params=pltpu.CompilerParams(use_tc_tiling_on_sc=True),
  )
  return run(field0, field1, field2)
